# Initial kernel scaffold; baseline (speedup 1.0000x reference)
import jax
import jax.numpy as jnp
from jax.experimental import pallas as pl


def kernel(tx_x, nh_x, tx_edge_index, nh_edge_index, belongs_edge_index, influences_edge_index, W_l_tx, W_r_tx, b_tx, W_l_nh, W_r_nh, b_nh, W_gat, a_src, a_dst, b_gat, W_sp, b_sp, W_tp, b_tp, W_g1, b_g1, W_g2, b_g2, W_t2n, b_t2n, W_n2t, b_n2t, W_out, b_out):
    raise NotImplementedError("write your pallas kernel here")


# speedup vs baseline: 2.9082x; 2.9082x over previous
import functools

import jax
import jax.numpy as jnp
from jax import lax
from jax.experimental import pallas as pl
from jax.experimental.pallas import tpu as pltpu
from jax.experimental.pallas import tpu_sc as plsc

N_TX, N_NH, H = 50000, 5000, 128
E_TT, E_NN = 800000, 80000

NC, NS, L = 2, 16, 16
NW = NC * NS

N_TXP = 53248
N_NHP = 5120
E_TTP = 802816
E_NNP = 81920
E_GATP = 86016
B_ROWS = 416

_mesh = functools.partial(
    plsc.VectorSubcoreMesh, core_axis_name="c", subcore_axis_name="s",
    num_cores=NC, num_subcores=NS)


def _zero_buf(ref, rows, cols):
    z = jnp.zeros((L,), jnp.float32)
    def body(i, _):
        for q in range(cols // L):
            ref[i, pl.ds(q * L, L)] = z
        return 0
    lax.fori_loop(0, rows, body, 0)


def _fill_buf(ref, rows, cols, val):
    v = jnp.full((L,), val, jnp.float32)
    def body(i, _):
        for q in range(cols // L):
            ref[i, pl.ds(q * L, L)] = v
        return 0
    lax.fori_loop(0, rows, body, 0)


def _a_body(tx0, tx1, tx2, tx3, src2, dst2, s0, s1, s2, s3, c0, c1,
            acc, sbuf, dbuf, rows, ones, zbuf, sem):
    c = lax.axis_index("c")
    t = lax.axis_index("s")
    stripe = N_TXP // NS
    zrows = 104

    _zero_buf(zbuf, zrows, 32)
    _fill_buf(ones, 128, 32, 1.0)

    def zero_acc():
        def zc(z, _):
            pltpu.sync_copy(zbuf, acc.at[pl.ds(t * stripe + z * zrows, zrows)])
            return 0
        lax.fori_loop(0, stripe // zrows, zc, 0)

    def feat_pass(tx_ref, out_ref):
        zero_acc()
        plsc.subcore_barrier()
        def chunk(r, _):
            base = t * 392 + r
            pltpu.sync_copy(src2.at[pl.ds(base, 1)], sbuf)
            pltpu.sync_copy(dst2.at[pl.ds(base, 1)], dbuf)
            pltpu.async_copy(tx_ref.at[sbuf.at[0]], rows, sem).wait()
            pltpu.sync_copy(rows, acc.at[dbuf.at[0]], add=True)
            return 0
        lax.fori_loop(0, 392, chunk, 0)
        plsc.subcore_barrier()
        pltpu.sync_copy(acc.at[pl.ds(t * stripe, stripe)],
                        out_ref.at[pl.ds(t * stripe, stripe)])
        plsc.subcore_barrier()

    def count_pass(half, out_ref):
        zero_acc()
        plsc.subcore_barrier()
        def chunk(r, _):
            base = half * (E_TTP // 256) + t * 196 + r
            pltpu.sync_copy(dst2.at[pl.ds(base, 1)], dbuf)
            pltpu.sync_copy(ones, acc.at[dbuf.at[0]], add=True)
            return 0
        lax.fori_loop(0, 196, chunk, 0)
        plsc.subcore_barrier()
        pltpu.sync_copy(acc.at[pl.ds(t * stripe, stripe)],
                        out_ref.at[pl.ds(t * stripe, stripe)])
        plsc.subcore_barrier()

    @pl.when(c == 0)
    def _():
        feat_pass(tx0, s0)
        feat_pass(tx1, s1)
        count_pass(0, c0)

    @pl.when(c == 1)
    def _():
        feat_pass(tx2, s2)
        feat_pass(tx3, s3)
        count_pass(1, c1)


def _sc_tx_agg(txt, src2, dst2):
    f32 = jnp.float32
    out = [jax.ShapeDtypeStruct((N_TXP, 32), f32)] * 6
    k = pl.kernel(
        _a_body, out_type=tuple(out), mesh=_mesh(),
        compiler_params=pltpu.CompilerParams(use_tc_tiling_on_sc=False),
        scratch_types=[
            pltpu.MemorySpace.VMEM_SHARED((N_TXP, 32), f32),
            pltpu.VMEM((1, 128), jnp.int32),
            pltpu.VMEM((1, 128), jnp.int32),
            pltpu.VMEM((128, 32), f32),
            pltpu.VMEM((128, 32), f32),
            pltpu.VMEM((104, 32), f32),
            pltpu.SemaphoreType.DMA,
        ])
    return k(txt[0], txt[1], txt[2], txt[3], src2, dst2)


def _b_body(s0, s1, s2, s3, c0, c1, x, wl, wr, b, wt2n, bt2n, wout,
            msg_ref, tho_ref):
    wlv = wl[...]
    S = jnp.dot(s0[...], wlv[0:32], preferred_element_type=jnp.float32)
    S += jnp.dot(s1[...], wlv[32:64], preferred_element_type=jnp.float32)
    S += jnp.dot(s2[...], wlv[64:96], preferred_element_type=jnp.float32)
    S += jnp.dot(s3[...], wlv[96:128], preferred_element_type=jnp.float32)
    cnt = c0[...][:, 0:1] + c1[...][:, 0:1]
    inv = 1.0 / jnp.maximum(cnt, 1.0)
    agg = S * inv
    txh = jax.nn.relu(
        agg + jnp.dot(x[...], wr[...], preferred_element_type=jnp.float32)
        + b[...][None, :])
    msg_ref[...] = jnp.dot(txh, wt2n[...],
                           preferred_element_type=jnp.float32) + bt2n[...][None, :]
    tho_ref[...] = jnp.dot(txh, wout[...], preferred_element_type=jnp.float32)


def _tc_tx_dense(sl, cl, x, wl, wr, b, wt2n, bt2n, wout):
    RB = 1024
    grid = (N_TXP // RB,)
    blk32 = pl.BlockSpec((RB, 32), lambda i: (i, 0))
    blk128 = pl.BlockSpec((RB, 128), lambda i: (i, 0))
    wspec = pl.BlockSpec((128, 128), lambda i: (0, 0))
    bspec = pl.BlockSpec((128,), lambda i: (0,))
    return pl.pallas_call(
        _b_body, grid=grid,
        in_specs=[blk32] * 6 + [blk128, wspec, wspec, bspec, wspec, bspec,
                                pl.BlockSpec((128, 1), lambda i: (0, 0))],
        out_specs=[blk128, pl.BlockSpec((RB, 1), lambda i: (i, 0))],
        out_shape=[jax.ShapeDtypeStruct((N_TXP, 128), jnp.float32),
                   jax.ShapeDtypeStruct((N_TXP, 1), jnp.float32)],
    )(*sl, *cl, x, wl, wr, b, wt2n, bt2n, wout)


def _c_body(nh_x, nsrc2, ndst2, txm, bdst2, sS0, sS1, cN0, cN1, aB0, aB1,
            accS, accC, accB, sbuf, dbuf, rows, ones, zbuf, zbufc, sem):
    c = lax.axis_index("c")
    t = lax.axis_index("s")
    w = c * NS + t
    stripe = N_NHP // NS

    _zero_buf(zbuf, 40, 128)
    _zero_buf(zbufc, 40, 32)
    _fill_buf(ones, 128, 32, 1.0)

    def zc(z, _):
        o = pl.ds(t * stripe + z * 40, 40)
        pltpu.sync_copy(zbuf, accS.at[o])
        pltpu.sync_copy(zbuf, accB.at[o])
        pltpu.sync_copy(zbufc, accC.at[o])
        return 0
    lax.fori_loop(0, stripe // 40, zc, 0)
    plsc.subcore_barrier()

    def nh_chunk(r, _):
        base = w * (E_NNP // 128 // NW) + r
        pltpu.sync_copy(nsrc2.at[pl.ds(base, 1)], sbuf)
        pltpu.sync_copy(ndst2.at[pl.ds(base, 1)], dbuf)
        pltpu.async_copy(nh_x.at[sbuf.at[0]], rows, sem).wait()
        pltpu.sync_copy(rows, accS.at[dbuf.at[0]], add=True)
        pltpu.sync_copy(ones, accC.at[dbuf.at[0]], add=True)
        return 0
    lax.fori_loop(0, E_NNP // 128 // NW, nh_chunk, 0)

    def bel_chunk(r, _):
        base = w * (B_ROWS // NW) + r
        pltpu.sync_copy(bdst2.at[pl.ds(base, 1)], dbuf)
        pltpu.sync_copy(txm.at[pl.ds(base * 128, 128)], rows)
        pltpu.sync_copy(rows, accB.at[dbuf.at[0]], add=True)
        return 0
    lax.fori_loop(0, B_ROWS // NW, bel_chunk, 0)

    plsc.subcore_barrier()
    sl = pl.ds(t * stripe, stripe)

    @pl.when(c == 0)
    def _():
        pltpu.sync_copy(accS.at[sl], sS0.at[sl])
        pltpu.sync_copy(accC.at[sl], cN0.at[sl])
        pltpu.sync_copy(accB.at[sl], aB0.at[sl])

    @pl.when(c == 1)
    def _():
        pltpu.sync_copy(accS.at[sl], sS1.at[sl])
        pltpu.sync_copy(accC.at[sl], cN1.at[sl])
        pltpu.sync_copy(accB.at[sl], aB1.at[sl])


def _sc_nh_agg(nh_x, nsrc2, ndst2, txm, bdst2):
    f32 = jnp.float32
    out = [jax.ShapeDtypeStruct((N_NHP, 128), f32),
           jax.ShapeDtypeStruct((N_NHP, 128), f32),
           jax.ShapeDtypeStruct((N_NHP, 32), f32),
           jax.ShapeDtypeStruct((N_NHP, 32), f32),
           jax.ShapeDtypeStruct((N_NHP, 128), f32),
           jax.ShapeDtypeStruct((N_NHP, 128), f32)]
    k = pl.kernel(
        _c_body, out_type=tuple(out), mesh=_mesh(),
        compiler_params=pltpu.CompilerParams(use_tc_tiling_on_sc=False),
        scratch_types=[
            pltpu.MemorySpace.VMEM_SHARED((N_NHP, 128), f32),
            pltpu.MemorySpace.VMEM_SHARED((N_NHP, 32), f32),
            pltpu.MemorySpace.VMEM_SHARED((N_NHP, 128), f32),
            pltpu.VMEM((1, 128), jnp.int32),
            pltpu.VMEM((1, 128), jnp.int32),
            pltpu.VMEM((128, 128), f32),
            pltpu.VMEM((128, 32), f32),
            pltpu.VMEM((40, 128), f32),
            pltpu.VMEM((40, 32), f32),
            pltpu.SemaphoreType.DMA,
        ])
    o = k(nh_x, nsrc2, ndst2, txm, bdst2)
    return (o[0], o[1]), (o[2], o[3]), (o[4], o[5])


def _d_body(sS0, sS1, cN0, cN1, aB0, aB1, nx, wl, wr, b, wgat, asrc, adst,
            wsp, bsp, h_ref, hs_ref, hd_ref, m_ref, sp_ref, agg_ref):
    S = sS0[...] + sS1[...]
    cnt = cN0[...][:, 0:1] + cN1[...][:, 0:1]
    inv = 1.0 / jnp.maximum(cnt, 1.0)
    nsp = jax.nn.relu(
        jnp.dot(S, wl[...], preferred_element_type=jnp.float32) * inv
        + jnp.dot(nx[...], wr[...], preferred_element_type=jnp.float32)
        + b[...][None, :])
    h = jnp.dot(nsp, wgat[...], preferred_element_type=jnp.float32)
    hs = jnp.dot(h, asrc[...][:, None], preferred_element_type=jnp.float32)
    hd = jnp.dot(h, adst[...][:, None], preferred_element_type=jnp.float32)
    ridx = lax.broadcasted_iota(jnp.int32, (N_NHP, 1), 0)
    maxS = jnp.max(jnp.where(ridx < N_NH, hs, -jnp.inf))
    M = hd + maxS
    m = jnp.maximum(M, 0.2 * M)
    h_ref[...] = h
    hs_ref[...] = hs.reshape(1, N_NHP)
    hd_ref[...] = hd.reshape(1, N_NHP)
    m_ref[...] = m.reshape(1, N_NHP)
    sp_ref[...] = jnp.dot(nsp, wsp[...],
                          preferred_element_type=jnp.float32) + bsp[...][None, :]
    agg_ref[...] = aB0[...] + aB1[...]


def _tc_nh_dense(sS, cN, aB, nx, wl, wr, b, wgat, asrc, adst, wsp, bsp):
    f32 = jnp.float32
    return pl.pallas_call(
        _d_body,
        out_shape=[jax.ShapeDtypeStruct((N_NHP, 128), f32),
                   jax.ShapeDtypeStruct((1, N_NHP), f32),
                   jax.ShapeDtypeStruct((1, N_NHP), f32),
                   jax.ShapeDtypeStruct((1, N_NHP), f32),
                   jax.ShapeDtypeStruct((N_NHP, 128), f32),
                   jax.ShapeDtypeStruct((N_NHP, 128), f32)],
    )(sS[0], sS[1], cN[0], cN[1], aB[0], aB[1], nx, wl, wr, b, wgat,
      asrc, adst, wsp, bsp)


def _e_body(h, hs, hd, m, gsrc2, gdst2, gA0, gA1,
            acc, hs_t, hd_t, m_t, sbuf, dbuf, rows, stage, exb, zbuf, sem):
    c = lax.axis_index("c")
    t = lax.axis_index("s")
    w = c * NS + t
    stripe = N_NHP // NS

    pltpu.sync_copy(hs.at[0], hs_t)
    pltpu.sync_copy(hd.at[0], hd_t)
    pltpu.sync_copy(m.at[0], m_t)
    _zero_buf(zbuf, 40, 144)
    def zc(z, _):
        pltpu.sync_copy(zbuf, acc.at[pl.ds(t * stripe + z * 40, 40)])
        return 0
    lax.fori_loop(0, stripe // 40, zc, 0)
    plsc.subcore_barrier()

    lane0 = lax.iota(jnp.int32, L) == 0

    def chunk(r, _):
        base = w * (E_GATP // 128 // NW) + r
        pltpu.sync_copy(gsrc2.at[pl.ds(base, 1)], sbuf)
        pltpu.sync_copy(gdst2.at[pl.ds(base, 1)], dbuf)
        pltpu.async_copy(h.at[sbuf.at[0]], rows, sem).wait()
        for g in range(128 // L):
            iv = sbuf[0, pl.ds(g * L, L)]
            dv = dbuf[0, pl.ds(g * L, L)]
            hsv = plsc.load_gather(hs_t, [iv])
            hdv = plsc.load_gather(hd_t, [dv])
            mv = plsc.load_gather(m_t, [dv])
            e = hsv + hdv
            e = jnp.maximum(e, 0.2 * e)
            exb[pl.ds(g * L, L)] = jnp.exp(e - mv)
        def edge(j, _):
            s = exb[pl.ds(j, L)][0]
            for q in range(8):
                stage[j, pl.ds(q * L, L)] = rows[j, pl.ds(q * L, L)] * s
            stage[j, pl.ds(128, L)] = jnp.where(lane0, s, 0.0)
            return 0
        lax.fori_loop(0, 128, edge, 0)
        pltpu.sync_copy(stage, acc.at[dbuf.at[0]], add=True)
        return 0
    lax.fori_loop(0, E_GATP // 128 // NW, chunk, 0)

    plsc.subcore_barrier()
    sl = pl.ds(t * stripe, stripe)

    @pl.when(c == 0)
    def _():
        pltpu.sync_copy(acc.at[sl], gA0.at[sl])

    @pl.when(c == 1)
    def _():
        pltpu.sync_copy(acc.at[sl], gA1.at[sl])


def _sc_gat(h, hs, hd, m, gsrc2, gdst2):
    f32 = jnp.float32
    out = [jax.ShapeDtypeStruct((N_NHP, 144), f32)] * 2
    k = pl.kernel(
        _e_body, out_type=tuple(out), mesh=_mesh(),
        compiler_params=pltpu.CompilerParams(
            use_tc_tiling_on_sc=False, needs_layout_passes=False),
        scratch_types=[
            pltpu.MemorySpace.VMEM_SHARED((N_NHP, 144), f32),
            pltpu.VMEM((N_NHP,), f32),
            pltpu.VMEM((N_NHP,), f32),
            pltpu.VMEM((N_NHP,), f32),
            pltpu.VMEM((1, 128), jnp.int32),
            pltpu.VMEM((1, 128), jnp.int32),
            pltpu.VMEM((128, 128), f32),
            pltpu.VMEM((128, 144), f32),
            pltpu.VMEM((144,), f32),
            pltpu.VMEM((40, 144), f32),
            pltpu.SemaphoreType.DMA,
        ])
    return k(h, hs, hd, m, gsrc2, gdst2)


def _f_body(gA0, gA1, sp, agg, bgat, wtp, btp, wg1, bg1, wg2, bg2,
            wn2t, bn2t, wout, bout, v_ref):
    g0 = gA0[...]
    g1 = gA1[...]
    num = g0[:, 0:128] + g1[:, 0:128]
    den = jnp.maximum(g0[:, 128:129] + g1[:, 128:129], 1e-30)
    nt = num / den + bgat[...][None, :]
    spv = sp[...]
    tp = jnp.dot(nt + agg[...], wtp[...],
                 preferred_element_type=jnp.float32) + btp[...][None, :]
    wg1v = wg1[...]
    r1 = jax.nn.relu(
        jnp.dot(spv, wg1v[0:128], preferred_element_type=jnp.float32)
        + jnp.dot(tp, wg1v[128:256], preferred_element_type=jnp.float32)
        + bg1[...][None, :])
    g = jax.nn.sigmoid(
        jnp.dot(r1, wg2[...], preferred_element_type=jnp.float32)
        + bg2[...][None, :])
    nout = jax.nn.relu(g * spv + (1.0 - g) * tp)
    n2t = jnp.dot(nout, wn2t[...],
                  preferred_element_type=jnp.float32) + bn2t[...][None, :]
    v_ref[...] = jnp.dot(n2t, wout[...],
                         preferred_element_type=jnp.float32) + bout[0]


def _tc_gate(gA, sp, agg, bgat, wtp, btp, wg1, bg1, wg2, bg2,
             wn2t, bn2t, wout, bout):
    return pl.pallas_call(
        _f_body,
        out_shape=jax.ShapeDtypeStruct((N_NHP, 1), jnp.float32),
    )(gA[0], gA[1], sp, agg, bgat, wtp, btp, wg1, bg1, wg2, bg2,
      wn2t, bn2t, wout, bout)


def _g_body(v, txh2, idx2, o2, v_t, ibuf, tbuf, obuf, z16_unused):
    c = lax.axis_index("c")
    t = lax.axis_index("s")
    w = c * NS + t
    pltpu.sync_copy(v, v_t)
    z16 = jnp.zeros((L,), jnp.int32)

    def chunk(r, _):
        base = w * (B_ROWS // NW) + r
        pltpu.sync_copy(idx2.at[pl.ds(base, 1)], ibuf)
        pltpu.sync_copy(txh2.at[pl.ds(base, 1)], tbuf)
        for g in range(128 // L):
            iv = ibuf[0, pl.ds(g * L, L)]
            vv = plsc.load_gather(v_t, [iv, z16])
            obuf[0, pl.ds(g * L, L)] = vv + tbuf[0, pl.ds(g * L, L)]
        pltpu.sync_copy(obuf, o2.at[pl.ds(base, 1)])
        return 0
    lax.fori_loop(0, B_ROWS // NW, chunk, 0)


def _sc_final(v, txh2, idx2):
    f32 = jnp.float32
    k = pl.kernel(
        _g_body, out_type=jax.ShapeDtypeStruct((B_ROWS, 128), f32),
        mesh=_mesh(),
        compiler_params=pltpu.CompilerParams(
            use_tc_tiling_on_sc=False, needs_layout_passes=False),
        scratch_types=[
            pltpu.VMEM((N_NHP, 1), f32),
            pltpu.VMEM((1, 128), jnp.int32),
            pltpu.VMEM((1, 128), f32),
            pltpu.VMEM((1, 128), f32),
            pltpu.VMEM((L,), jnp.int32),
        ])
    return k(v, txh2, idx2)


def _pad1(x, n, val):
    return jnp.concatenate(
        [x, jnp.full((n - x.shape[0],), val, x.dtype)])


def kernel(tx_x, nh_x, tx_edge_index, nh_edge_index, belongs_edge_index,
           influences_edge_index, W_l_tx, W_r_tx, b_tx, W_l_nh, W_r_nh, b_nh,
           W_gat, a_src, a_dst, b_gat, W_sp, b_sp, W_tp, b_tp, W_g1, b_g1,
           W_g2, b_g2, W_t2n, b_t2n, W_n2t, b_n2t, W_out, b_out):
    f32 = jnp.float32
    tx_x_p = jnp.concatenate(
        [tx_x, jnp.zeros((N_TXP - N_TX, 128), f32)])
    txt = jnp.transpose(tx_x_p.reshape(N_TXP, 4, 32), (1, 0, 2))
    txt = [txt[0], txt[1], txt[2], txt[3]]
    src2 = _pad1(tx_edge_index[0], E_TTP, 0).reshape(E_TTP // 128, 128)
    dst2 = _pad1(tx_edge_index[1], E_TTP, N_TX).reshape(E_TTP // 128, 128)
    nsrc2 = _pad1(nh_edge_index[0], E_NNP, 0).reshape(E_NNP // 128, 128)
    ndst2 = _pad1(nh_edge_index[1], E_NNP, N_NH).reshape(E_NNP // 128, 128)
    loop = jnp.arange(N_NH, dtype=jnp.int32)
    gsrc2 = _pad1(jnp.concatenate([nh_edge_index[0], loop]),
                  E_GATP, 0).reshape(E_GATP // 128, 128)
    gdst2 = _pad1(jnp.concatenate([nh_edge_index[1], loop]),
                  E_GATP, N_NH).reshape(E_GATP // 128, 128)
    bdst2 = _pad1(belongs_edge_index[1], N_TXP, N_NH).reshape(B_ROWS, 128)
    iidx2 = _pad1(influences_edge_index[1], N_TXP, 0).reshape(B_ROWS, 128)
    nh_x_p = jnp.concatenate([nh_x, jnp.zeros((N_NHP - N_NH, 128), f32)])

    s_all = _sc_tx_agg(txt, src2, dst2)
    tx_msg, txh = _tc_tx_dense(s_all[0:4], s_all[4:6], tx_x_p,
                               W_l_tx, W_r_tx, b_tx, W_t2n, b_t2n, W_out)
    sS, cN, aB = _sc_nh_agg(nh_x, nsrc2, ndst2, tx_msg, bdst2)
    h, hs, hd, m, sp, agg = _tc_nh_dense(
        sS, cN, aB, nh_x_p, W_l_nh, W_r_nh, b_nh, W_gat, a_src, a_dst,
        W_sp, b_sp)
    gA = _sc_gat(h, hs, hd, m, gsrc2, gdst2)
    v = _tc_gate(gA, sp, agg, b_gat, W_tp, b_tp, W_g1, b_g1, W_g2, b_g2,
                 W_n2t, b_n2t, W_out, b_out)
    o2 = _sc_final(v, txh.reshape(B_ROWS, 128), iidx2)
    return o2.reshape(N_TXP)[:N_TX]
